# baseline (speedup 1.0000x reference)
import jax
import jax.numpy as jnp
from jax import lax
from jax.experimental import pallas as pl
from jax.experimental.pallas import tpu as pltpu
from jax.experimental.pallas import tpu_sc as plsc

B = 4
V = 100000
F = 200000

NC = 2
NS = 16
L = 16
W = 8

RPW = 6400
V_PAD = NS * RPW
NCH = 800
NJ = 98
F_PAD = NS * NJ * 128


def _rsqrt_or_one(n2):
    bits = plsc.bitcast(n2, jnp.int32)
    seed = jnp.int32(0x5F3759DF) - lax.shift_right_logical(bits, 1)
    y = plsc.bitcast(seed, jnp.float32)
    for _ in range(3):
        y = y * (1.5 - 0.5 * n2 * y * y)
    return jnp.where(n2 < 1e-10, jnp.float32(1.0), y)


def _sc_body(verts_h, vit_h, zeros_h, out_h,
             acc, idx0, idx1, idx2, viraw,
             vb00, vb01, vb02, vb10, vb11, vb12,
             nb0, nb1, stage, outb,
             gsem0, gsem1, ssem0, ssem1):
    cid = lax.axis_index("c")
    sid = lax.axis_index("s")
    vstart = sid * RPW
    iota = lax.iota(jnp.int32, L)
    col = [jnp.full((L,), c, jnp.int32) for c in range(3)]
    zvec = jnp.zeros((L,), jnp.float32)

    idx = (idx0, idx1, idx2)
    vb = ((vb00, vb01, vb02), (vb10, vb11, vb12))
    nb = (nb0, nb1)
    gsem = (gsem0, gsem1)
    ssem = (ssem0, ssem1)

    def extract_step(ch, carry):
        pltpu.sync_copy(vit_h.at[sid, pl.ds(ch * (7 * 128), 7 * 128)], viraw)
        for blk in range(7):
            j = ch * 7 + blk
            jvec = jnp.full((L,), 0, jnp.int32) + j
            for g in range(128 // L):
                rows = iota + blk * 128 + g * L
                dcol = iota + g * L
                for k in range(3):
                    v = plsc.load_gather(viraw, [rows, col[k]])
                    plsc.store_scatter(idx[k], [jvec, dcol], v)
        return carry

    lax.fori_loop(0, NJ // 7, extract_step, 0)

    for s in range(2):
        for g in range(128 // L):
            for c in range(3, W):
                plsc.store_scatter(
                    nb[s], [iota + g * L, jnp.full((L,), c, jnp.int32)], zvec)

    def fire_gather(b, j, s):
        for k in range(3):
            pltpu.async_copy(verts_h.at[b].at[idx[k].at[j]], vb[s][k], gsem[s])

    def wait_gather(b, j, s):
        for k in range(3):
            pltpu.make_async_copy(
                verts_h.at[b].at[idx[k].at[j]], vb[s][k], gsem[s]).wait()

    def fire_scatter(j, s):
        for k in range(3):
            pltpu.async_copy(nb[s], acc.at[idx[k].at[j]], ssem[s], add=True)

    def wait_scatter(j, s):
        for k in range(3):
            pltpu.make_async_copy(nb[s], acc.at[idx[k].at[j]], ssem[s]).wait()

    def compute(s):
        for g in range(128 // L):
            rows = iota + g * L
            a0 = plsc.load_gather(vb[s][0], [rows, col[0]])
            a1 = plsc.load_gather(vb[s][0], [rows, col[1]])
            a2 = plsc.load_gather(vb[s][0], [rows, col[2]])
            b0 = plsc.load_gather(vb[s][1], [rows, col[0]])
            b1 = plsc.load_gather(vb[s][1], [rows, col[1]])
            b2 = plsc.load_gather(vb[s][1], [rows, col[2]])
            c0 = plsc.load_gather(vb[s][2], [rows, col[0]])
            c1 = plsc.load_gather(vb[s][2], [rows, col[1]])
            c2 = plsc.load_gather(vb[s][2], [rows, col[2]])
            e1x, e1y, e1z = b0 - a0, b1 - a1, b2 - a2
            e2x, e2y, e2z = c0 - a0, c1 - a1, c2 - a2
            nx = e1y * e2z - e1z * e2y
            ny = e1z * e2x - e1x * e2z
            nz = e1x * e2y - e1y * e2x
            sc = _rsqrt_or_one(nx * nx + ny * ny + nz * nz)
            plsc.store_scatter(nb[s], [rows, col[0]], nx * sc)
            plsc.store_scatter(nb[s], [rows, col[1]], ny * sc)
            plsc.store_scatter(nb[s], [rows, col[2]], nz * sc)

    for t in range(2):
        b = cid * 2 + t
        pltpu.sync_copy(zeros_h.at[pl.ds(vstart, RPW)],
                        acc.at[pl.ds(vstart, RPW)])
        plsc.subcore_barrier()

        fire_gather(b, 0, 0)

        def pair_step(jj, carry):
            for s in range(2):
                j = 2 * jj + s

                @pl.when(j + 1 < NJ)
                def _():
                    fire_gather(b, j + 1, 1 - s)

                wait_gather(b, j, s)

                @pl.when(j >= 2)
                def _():
                    wait_scatter(j - 2, s)

                compute(s)
                fire_scatter(j, s)
            return carry

        lax.fori_loop(0, NJ // 2, pair_step, 0)
        wait_scatter(NJ - 2, 0)
        wait_scatter(NJ - 1, 1)
        plsc.subcore_barrier()

        def norm_step(j, carry):
            rows = iota + j * L
            x = plsc.load_gather(stage, [rows, col[0]])
            y = plsc.load_gather(stage, [rows, col[1]])
            z = plsc.load_gather(stage, [rows, col[2]])
            sc = _rsqrt_or_one(x * x + y * y + z * z)
            plsc.store_scatter(outb, [rows, col[0]], x * sc)
            plsc.store_scatter(outb, [rows, col[1]], y * sc)
            plsc.store_scatter(outb, [rows, col[2]], z * sc)
            return carry

        for k in range(RPW // NCH):
            pltpu.sync_copy(acc.at[pl.ds(vstart + k * NCH, NCH)], stage)
            lax.fori_loop(0, NCH // L, norm_step, 0)
            pltpu.sync_copy(outb, out_h.at[b, pl.ds(vstart + k * NCH, NCH)])
        plsc.subcore_barrier()


@jax.jit
def kernel(verts, vi):
    verts_pad = jnp.zeros((B, V_PAD, W), jnp.float32)
    verts_pad = verts_pad.at[:, :V, :3].set(verts)
    vit = jnp.full((F_PAD, 3), V, jnp.int32)
    vit = vit.at[:F].set(vi).reshape(NS, NJ * 128, 3)
    zeros = jnp.zeros((V_PAD, W), jnp.float32)

    mesh = plsc.VectorSubcoreMesh(core_axis_name="c", subcore_axis_name="s")
    run = pl.kernel(
        _sc_body,
        out_type=jax.ShapeDtypeStruct((B, V_PAD, 3), jnp.float32),
        mesh=mesh,
        compiler_params=pltpu.CompilerParams(
            needs_layout_passes=False, use_tc_tiling_on_sc=False),
        scratch_types=[
            pltpu.VMEM_SHARED((V_PAD, W), jnp.float32),
            pltpu.VMEM((NJ, 128), jnp.int32),
            pltpu.VMEM((NJ, 128), jnp.int32),
            pltpu.VMEM((NJ, 128), jnp.int32),
            pltpu.VMEM((7 * 128, 3), jnp.int32),
            pltpu.VMEM((128, W), jnp.float32),
            pltpu.VMEM((128, W), jnp.float32),
            pltpu.VMEM((128, W), jnp.float32),
            pltpu.VMEM((128, W), jnp.float32),
            pltpu.VMEM((128, W), jnp.float32),
            pltpu.VMEM((128, W), jnp.float32),
            pltpu.VMEM((128, W), jnp.float32),
            pltpu.VMEM((128, W), jnp.float32),
            pltpu.VMEM((NCH, W), jnp.float32),
            pltpu.VMEM((NCH, 3), jnp.float32),
            pltpu.SemaphoreType.DMA,
            pltpu.SemaphoreType.DMA,
            pltpu.SemaphoreType.DMA,
            pltpu.SemaphoreType.DMA,
        ],
    )
    out = run(verts_pad, vit, zeros)
    return out[:, :V, :]


# speedup vs baseline: 1.2140x; 1.2140x over previous
import jax
import jax.numpy as jnp
from jax import lax
from jax.experimental import pallas as pl
from jax.experimental.pallas import tpu as pltpu
from jax.experimental.pallas import tpu_sc as plsc

B = 4
V = 100000
F = 200000

NC = 2
NS = 16
L = 16
W = 8

RPW = 6400
V_PAD = NS * RPW
NCH = 800
NJ = 98
F_PAD = NS * NJ * 128


def _rsqrt_or_one(n2):
    bits = plsc.bitcast(n2, jnp.int32)
    seed = jnp.int32(0x5F3759DF) - lax.shift_right_logical(bits, 1)
    y = plsc.bitcast(seed, jnp.float32)
    for _ in range(3):
        y = y * (1.5 - 0.5 * n2 * y * y)
    return jnp.where(n2 < 1e-10, jnp.float32(1.0), y)


def _sc_body(verts_h, vit_h, out_h,
             acc, idx, vg0, vg1, ng0, ng1, stage, outb,
             gsem0, gsem1, ssem0, ssem1):
    cid = lax.axis_index("c")
    sid = lax.axis_index("s")
    vstart = sid * RPW
    iota = lax.iota(jnp.int32, L)
    col = [jnp.full((L,), c, jnp.int32) for c in range(3)]
    zvec = jnp.zeros((L,), jnp.float32)

    vg = (vg0, vg1)
    ng = (ng0, ng1)
    gsem = (gsem0, gsem1)
    ssem = (ssem0, ssem1)

    pltpu.sync_copy(vit_h.at[sid], idx)

    for s in range(2):
        for g in range(384 // L):
            for c in range(3, W):
                plsc.store_scatter(
                    ng[s], [iota + g * L, jnp.full((L,), c, jnp.int32)], zvec)

    def zero_stage():
        def zs(g, carry):
            rows = lax.shift_right_logical(iota, 3) + g * 2
            cols = jnp.bitwise_and(iota, 7)
            plsc.store_scatter(stage, [rows, cols], zvec)
            return carry
        lax.fori_loop(0, NCH // 2, zs, 0)

    def fire_gather(b, j, s):
        for m in range(3):
            pltpu.async_copy(verts_h.at[b].at[idx.at[3 * j + m]],
                             vg[s].at[pl.ds(m * 128, 128)], gsem[s])

    def wait_gather(b, j, s):
        for m in range(3):
            pltpu.make_async_copy(verts_h.at[b].at[idx.at[3 * j + m]],
                                  vg[s].at[pl.ds(m * 128, 128)],
                                  gsem[s]).wait()

    def fire_scatter(j, s):
        for m in range(3):
            pltpu.async_copy(ng[s].at[pl.ds(m * 128, 128)],
                             acc.at[idx.at[3 * j + m]], ssem[s], add=True)

    def wait_scatter(j, s):
        for m in range(3):
            pltpu.make_async_copy(ng[s].at[pl.ds(m * 128, 128)],
                                  acc.at[idx.at[3 * j + m]], ssem[s]).wait()

    def compute(s):
        for g in range(128 // L):
            p = [iota * 3 + (48 * g + j) for j in range(3)]
            a0 = plsc.load_gather(vg[s], [p[0], col[0]])
            a1 = plsc.load_gather(vg[s], [p[0], col[1]])
            a2 = plsc.load_gather(vg[s], [p[0], col[2]])
            b0 = plsc.load_gather(vg[s], [p[1], col[0]])
            b1 = plsc.load_gather(vg[s], [p[1], col[1]])
            b2 = plsc.load_gather(vg[s], [p[1], col[2]])
            c0 = plsc.load_gather(vg[s], [p[2], col[0]])
            c1 = plsc.load_gather(vg[s], [p[2], col[1]])
            c2 = plsc.load_gather(vg[s], [p[2], col[2]])
            e1x, e1y, e1z = b0 - a0, b1 - a1, b2 - a2
            e2x, e2y, e2z = c0 - a0, c1 - a1, c2 - a2
            nx = e1y * e2z - e1z * e2y
            ny = e1z * e2x - e1x * e2z
            nz = e1x * e2y - e1y * e2x
            sc = _rsqrt_or_one(nx * nx + ny * ny + nz * nz)
            vx, vy, vz = nx * sc, ny * sc, nz * sc
            for j in range(3):
                plsc.store_scatter(ng[s], [p[j], col[0]], vx)
                plsc.store_scatter(ng[s], [p[j], col[1]], vy)
                plsc.store_scatter(ng[s], [p[j], col[2]], vz)

    for t in range(2):
        b = cid * 2 + t
        zero_stage()
        for k in range(RPW // NCH):
            pltpu.sync_copy(stage, acc.at[pl.ds(vstart + k * NCH, NCH)])
        plsc.subcore_barrier()

        fire_gather(b, 0, 0)

        def pair_step(jj, carry):
            for s in range(2):
                j = 2 * jj + s

                @pl.when(j + 1 < NJ)
                def _():
                    fire_gather(b, j + 1, 1 - s)

                wait_gather(b, j, s)

                @pl.when(j >= 2)
                def _():
                    wait_scatter(j - 2, s)

                compute(s)
                fire_scatter(j, s)
            return carry

        lax.fori_loop(0, NJ // 2, pair_step, 0)
        wait_scatter(NJ - 2, 0)
        wait_scatter(NJ - 1, 1)
        plsc.subcore_barrier()

        def norm_step(j, carry):
            rows = iota + j * L
            x = plsc.load_gather(stage, [rows, col[0]])
            y = plsc.load_gather(stage, [rows, col[1]])
            z = plsc.load_gather(stage, [rows, col[2]])
            sc = _rsqrt_or_one(x * x + y * y + z * z)
            plsc.store_scatter(outb, [rows, col[0]], x * sc)
            plsc.store_scatter(outb, [rows, col[1]], y * sc)
            plsc.store_scatter(outb, [rows, col[2]], z * sc)
            return carry

        for k in range(RPW // NCH):
            @pl.when(vstart + k * NCH < V)
            def _():
                pltpu.sync_copy(acc.at[pl.ds(vstart + k * NCH, NCH)], stage)
                lax.fori_loop(0, NCH // L, norm_step, 0)
                pltpu.sync_copy(outb, out_h.at[b, pl.ds(vstart + k * NCH, NCH)])
        plsc.subcore_barrier()


@jax.jit
def kernel(verts, vi):
    verts_pad = jnp.zeros((B, V_PAD, W), jnp.float32)
    verts_pad = verts_pad.at[:, :V, :3].set(verts)
    vit = jnp.full((F_PAD, 3), V, jnp.int32)
    vit = vit.at[:F].set(vi).reshape(NS, 3 * NJ, 128)

    mesh = plsc.VectorSubcoreMesh(core_axis_name="c", subcore_axis_name="s")
    run = pl.kernel(
        _sc_body,
        out_type=jax.ShapeDtypeStruct((B, V, 3), jnp.float32),
        mesh=mesh,
        compiler_params=pltpu.CompilerParams(
            needs_layout_passes=False, use_tc_tiling_on_sc=False),
        scratch_types=[
            pltpu.VMEM_SHARED((V_PAD, W), jnp.float32),
            pltpu.VMEM((3 * NJ, 128), jnp.int32),
            pltpu.VMEM((384, W), jnp.float32),
            pltpu.VMEM((384, W), jnp.float32),
            pltpu.VMEM((384, W), jnp.float32),
            pltpu.VMEM((384, W), jnp.float32),
            pltpu.VMEM((NCH, W), jnp.float32),
            pltpu.VMEM((NCH, 3), jnp.float32),
            pltpu.SemaphoreType.DMA,
            pltpu.SemaphoreType.DMA,
            pltpu.SemaphoreType.DMA,
            pltpu.SemaphoreType.DMA,
        ],
    )
    return run(verts_pad, vit)


# speedup vs baseline: 1.2699x; 1.0460x over previous
import jax
import jax.numpy as jnp
from jax import lax
from jax.experimental import pallas as pl
from jax.experimental.pallas import tpu as pltpu
from jax.experimental.pallas import tpu_sc as plsc

B = 4
V = 100000
F = 200000

NC = 2
NS = 16
L = 16
W = 8

RPW = 6400
V_PAD = NS * RPW
NCH = 800
NJ = 98
F_PAD = NS * NJ * 128


def _rsqrt_or_one(n2):
    bits = plsc.bitcast(n2, jnp.int32)
    seed = jnp.int32(0x5F3759DF) - lax.shift_right_logical(bits, 1)
    y = plsc.bitcast(seed, jnp.float32)
    for _ in range(3):
        y = y * (1.5 - 0.5 * n2 * y * y)
    return jnp.where(n2 < 1e-10, jnp.float32(1.0), y)


def _sc_body(verts_h, vit_h, zeros_h, out_h,
             acc, idx0, idx1, idx2,
             vb00, vb01, vb02, vb10, vb11, vb12,
             nb0, nb1, stage, outb,
             gsem0, gsem1, ssem0, ssem1):
    cid = lax.axis_index("c")
    sid = lax.axis_index("s")
    vstart = sid * RPW
    iota = lax.iota(jnp.int32, L)
    col = [jnp.full((L,), c, jnp.int32) for c in range(3)]
    zvec = jnp.zeros((L,), jnp.float32)

    idx = (idx0, idx1, idx2)
    vb = ((vb00, vb01, vb02), (vb10, vb11, vb12))
    nb = (nb0, nb1)
    gsem = (gsem0, gsem1)
    ssem = (ssem0, ssem1)

    for k in range(3):
        pltpu.sync_copy(vit_h.at[k, sid], idx[k])

    for s in range(2):
        for g in range(128 // L):
            for c in range(3, W):
                plsc.store_scatter(
                    nb[s], [iota + g * L, jnp.full((L,), c, jnp.int32)], zvec)

    def fire_gather(b, j, s):
        for k in range(3):
            pltpu.async_copy(verts_h.at[b].at[idx[k].at[j]], vb[s][k], gsem[s])

    def wait_gather(b, j, s):
        for k in range(3):
            pltpu.make_async_copy(
                verts_h.at[b].at[idx[k].at[j]], vb[s][k], gsem[s]).wait()

    def fire_scatter(j, s):
        for k in range(3):
            pltpu.async_copy(nb[s], acc.at[idx[k].at[j]], ssem[s], add=True)

    def wait_scatter(j, s):
        for k in range(3):
            pltpu.make_async_copy(nb[s], acc.at[idx[k].at[j]], ssem[s]).wait()

    def compute(s):
        for g in range(128 // L):
            rows = iota + g * L
            a0 = plsc.load_gather(vb[s][0], [rows, col[0]])
            a1 = plsc.load_gather(vb[s][0], [rows, col[1]])
            a2 = plsc.load_gather(vb[s][0], [rows, col[2]])
            b0 = plsc.load_gather(vb[s][1], [rows, col[0]])
            b1 = plsc.load_gather(vb[s][1], [rows, col[1]])
            b2 = plsc.load_gather(vb[s][1], [rows, col[2]])
            c0 = plsc.load_gather(vb[s][2], [rows, col[0]])
            c1 = plsc.load_gather(vb[s][2], [rows, col[1]])
            c2 = plsc.load_gather(vb[s][2], [rows, col[2]])
            e1x, e1y, e1z = b0 - a0, b1 - a1, b2 - a2
            e2x, e2y, e2z = c0 - a0, c1 - a1, c2 - a2
            nx = e1y * e2z - e1z * e2y
            ny = e1z * e2x - e1x * e2z
            nz = e1x * e2y - e1y * e2x
            sc = _rsqrt_or_one(nx * nx + ny * ny + nz * nz)
            plsc.store_scatter(nb[s], [rows, col[0]], nx * sc)
            plsc.store_scatter(nb[s], [rows, col[1]], ny * sc)
            plsc.store_scatter(nb[s], [rows, col[2]], nz * sc)

    for t in range(2):
        b = cid * 2 + t
        pltpu.sync_copy(zeros_h.at[pl.ds(vstart, RPW)],
                        acc.at[pl.ds(vstart, RPW)])
        plsc.subcore_barrier()

        fire_gather(b, 0, 0)

        def pair_step(jj, carry):
            for s in range(2):
                j = 2 * jj + s

                @pl.when(j + 1 < NJ)
                def _():
                    fire_gather(b, j + 1, 1 - s)

                wait_gather(b, j, s)

                @pl.when(j >= 2)
                def _():
                    wait_scatter(j - 2, s)

                compute(s)
                fire_scatter(j, s)
            return carry

        lax.fori_loop(0, NJ // 2, pair_step, 0)
        wait_scatter(NJ - 2, 0)
        wait_scatter(NJ - 1, 1)
        plsc.subcore_barrier()

        def norm_step(j, carry):
            rows = iota + j * L
            x = plsc.load_gather(stage, [rows, col[0]])
            y = plsc.load_gather(stage, [rows, col[1]])
            z = plsc.load_gather(stage, [rows, col[2]])
            sc = _rsqrt_or_one(x * x + y * y + z * z)
            plsc.store_scatter(outb, [rows, col[0]], x * sc)
            plsc.store_scatter(outb, [rows, col[1]], y * sc)
            plsc.store_scatter(outb, [rows, col[2]], z * sc)
            return carry

        for k in range(RPW // NCH):
            pltpu.sync_copy(acc.at[pl.ds(vstart + k * NCH, NCH)], stage)
            lax.fori_loop(0, NCH // L, norm_step, 0)
            pltpu.sync_copy(outb, out_h.at[b, pl.ds(vstart + k * NCH, NCH)])
        plsc.subcore_barrier()


@jax.jit
def kernel(verts, vi):
    verts_pad = jnp.zeros((B, V_PAD, W), jnp.float32)
    verts_pad = verts_pad.at[:, :V, :3].set(verts)
    vit = jnp.full((3, F_PAD), V, jnp.int32)
    vit = vit.at[:, :F].set(vi.T).reshape(3, NS, NJ, 128)
    zeros = jnp.zeros((V_PAD, W), jnp.float32)

    mesh = plsc.VectorSubcoreMesh(core_axis_name="c", subcore_axis_name="s")
    run = pl.kernel(
        _sc_body,
        out_type=jax.ShapeDtypeStruct((B, V_PAD, 3), jnp.float32),
        mesh=mesh,
        compiler_params=pltpu.CompilerParams(
            needs_layout_passes=False, use_tc_tiling_on_sc=False),
        scratch_types=[
            pltpu.VMEM_SHARED((V_PAD, W), jnp.float32),
            pltpu.VMEM((NJ, 128), jnp.int32),
            pltpu.VMEM((NJ, 128), jnp.int32),
            pltpu.VMEM((NJ, 128), jnp.int32),
            pltpu.VMEM((128, W), jnp.float32),
            pltpu.VMEM((128, W), jnp.float32),
            pltpu.VMEM((128, W), jnp.float32),
            pltpu.VMEM((128, W), jnp.float32),
            pltpu.VMEM((128, W), jnp.float32),
            pltpu.VMEM((128, W), jnp.float32),
            pltpu.VMEM((128, W), jnp.float32),
            pltpu.VMEM((128, W), jnp.float32),
            pltpu.VMEM((NCH, W), jnp.float32),
            pltpu.VMEM((NCH, 3), jnp.float32),
            pltpu.SemaphoreType.DMA,
            pltpu.SemaphoreType.DMA,
            pltpu.SemaphoreType.DMA,
            pltpu.SemaphoreType.DMA,
        ],
    )
    out = run(verts_pad, vit, zeros)
    return out[:, :V, :]
